# Initial kernel scaffold; baseline (speedup 1.0000x reference)
import jax
import jax.numpy as jnp
from jax.experimental import pallas as pl


def kernel(memory, node_ids, timestamps, time_w, time_b, msg_W, msg_b, rnn_Wih, rnn_Whh, rnn_b, dec_W1, dec_b1, dec_W2, dec_b2, dec_W3, dec_b3):
    raise NotImplementedError("write your pallas kernel here")


# speedup vs baseline: 3.1203x; 3.1203x over previous
import functools

import jax
import jax.numpy as jnp
from jax import lax
from jax.experimental import pallas as pl
from jax.experimental.pallas import tpu as pltpu
from jax.experimental.pallas import tpu_sc as plsc

N = 1000000
D = 32
B = 16384

NC = 2
NS = 16
NW = NC * NS
L = 16

TCOLS = N // 128
TAIL = TCOLS * 128
NTAIL = N - TAIL
TC_BASE = TCOLS // NW
TC_REM = TCOLS % NW
CHT = 5
CHN = CHT * 128
OWN_SZ = (TC_BASE + 2) * 128
IDS_BLK = 2048
STG = 512

_mesh = plsc.VectorSubcoreMesh(
    core_axis_name="c", subcore_axis_name="s", num_cores=NC, num_subcores=NS)
_params = pltpu.CompilerParams(needs_layout_passes=False)


def _wrange(wid):
    tc0 = wid * TC_BASE + jnp.minimum(wid, TC_REM)
    tc1 = (wid + 1) * TC_BASE + jnp.minimum(wid + 1, TC_REM)
    return tc0 * 128, (tc1 - tc0) * 128


def _compact(ids_hbm, idsb_v, idl_v, posl_v, lo, span):
    iota = lax.iota(jnp.int32, L)

    def blk(b, cnt):
        pltpu.sync_copy(ids_hbm.at[pl.ds(b * IDS_BLK, IDS_BLK)], idsb_v)

        def vec(i, cnt):
            ids = idsb_v[pl.ds(i * L, L)]
            pos = iota + (b * IDS_BLK + i * L)
            m = (ids >= lo) & (ids < lo + span)
            plsc.store_compressed(idl_v.at[pl.ds(cnt, L)], ids, mask=m)
            plsc.store_compressed(posl_v.at[pl.ds(cnt, L)], pos, mask=m)
            return cnt + jnp.sum(jnp.where(m, 1, 0))

        return lax.fori_loop(0, IDS_BLK // L, vec, cnt)

    cnt = lax.fori_loop(0, B // IDS_BLK, blk, jnp.int32(0))
    idl_v[pl.ds(cnt, L)] = jnp.full((L,), -1, jnp.int32)
    return cnt


@functools.partial(
    pl.kernel,
    mesh=_mesh,
    out_type=jax.ShapeDtypeStruct((B * D,), jnp.float32),
    compiler_params=_params,
    scratch_types=[
        pltpu.VMEM((IDS_BLK,), jnp.int32),
        pltpu.VMEM((B + L,), jnp.int32),
        pltpu.VMEM((B + L,), jnp.int32),
        pltpu.VMEM((STG,), jnp.float32),
        pltpu.VMEM((D, CHN), jnp.float32),
        pltpu.VMEM((D, CHN), jnp.float32),
        pltpu.VMEM((D, 128), jnp.float32),
        pltpu.VMEM((D, NTAIL), jnp.float32),
        pltpu.SemaphoreType.DMA,
        pltpu.SemaphoreType.DMA,
        pltpu.SemaphoreType.DMA,
    ],
)
def _sc_gather(mem_hbm, ids_hbm, tail_hbm, h_hbm, idsb_v, idl_v, posl_v,
               stg_v, cb0_v, cb1_v, rb_v, tb_v, sem_a, sem_b, sem_h):
    wid = lax.axis_index("s") * NC + lax.axis_index("c")
    lo, span = _wrange(wid)
    iota = lax.iota(jnp.int32, L)
    span_t = jnp.where(wid == NW - 1, span + NTAIL, span)
    cnt = _compact(ids_hbm, idsb_v, idl_v, posl_v, lo, span_t)
    nvl = (cnt + L - 1) // L

    def extract(bufview, clo, csz):
        def lvec(k, _):
            idv = idl_v[pl.ds(k * L, L)]
            posv = posl_v[pl.ds(k * L, L)]
            m = (idv >= clo) & (idv < clo + csz)
            nm = jnp.sum(jnp.where(m, 1, 0))

            @pl.when(nm > 0)
            def _():
                for l in range(L):
                    idx = idv[l]
                    sel = (idx >= clo) & (idx < clo + csz)

                    @pl.when(sel)
                    def _():
                        col = jnp.full((L,), idx - clo, jnp.int32)
                        v0 = plsc.load_gather(bufview, [iota, col])
                        v1 = plsc.load_gather(bufview, [iota + L, col])
                        stg_v[pl.ds(l * 2 * L, L)] = v0
                        stg_v[pl.ds(l * 2 * L + L, L)] = v1
                        pltpu.async_copy(
                            stg_v.at[pl.ds(l * 2 * L, 2 * L)],
                            h_hbm.at[pl.ds(posv[l] * D, D)],
                            sem_h,
                        )

                def drain(_k, _x):
                    pltpu.make_async_copy(
                        stg_v.at[pl.ds(0, D)], h_hbm.at[pl.ds(0, D)], sem_h
                    ).wait()
                    return 0

                lax.fori_loop(0, nm, drain, 0)

            return 0

        lax.fori_loop(0, nvl, lvec, 0)

    nch = span // CHN
    nrem = (span - nch * CHN) // 128

    def cin(c, buf, sem):
        return pltpu.make_async_copy(
            mem_hbm.at[:, pl.ds(lo + c * CHN, CHN)], buf, sem)

    @pl.when(nch > 0)
    def _():
        cin(0, cb0_v, sem_a).start()

        def body(c, _):
            even = c % 2 == 0

            @pl.when(even)
            def _():
                @pl.when(c + 1 < nch)
                def _():
                    cin(c + 1, cb1_v, sem_b).start()

                cin(c, cb0_v, sem_a).wait()
                extract(cb0_v, lo + c * CHN, CHN)

            @pl.when(~even)
            def _():
                @pl.when(c + 1 < nch)
                def _():
                    cin(c + 1, cb0_v, sem_a).start()

                cin(c, cb1_v, sem_b).wait()
                extract(cb1_v, lo + c * CHN, CHN)

            return 0

        lax.fori_loop(0, nch, body, 0)

    def rem_body(r, _):
        off = lo + nch * CHN + r * 128
        pltpu.async_copy(
            mem_hbm.at[:, pl.ds(off, 128)], rb_v, sem_a).wait()
        extract(rb_v, off, 128)
        return 0

    lax.fori_loop(0, nrem, rem_body, 0)

    @pl.when(wid == NW - 1)
    def _():
        pltpu.sync_copy(tail_hbm, tb_v)
        extract(tb_v, TAIL, NTAIL)


def _tc_body(h_ref, ts_ref, tw_ref, tb_ref, mw_ref, mb_ref, wih_ref, whh_ref,
             rb_ref, w1_ref, b1_ref, w2_ref, b2_ref, w3_ref, b3_ref,
             newh_ref, score_ref):
    h = h_ref[...]
    te = jnp.cos(ts_ref[...] * tw_ref[...] + tb_ref[...])
    msg = jnp.maximum(
        h @ mw_ref[0:D, :] + te @ mw_ref[D:2 * D, :] + mb_ref[...], 0.0)
    nh = jnp.tanh(msg @ wih_ref[...] + h @ whh_ref[...] + rb_ref[...])
    newh_ref[...] = nh
    x = jnp.maximum(
        h @ w1_ref[0:D, :] + nh @ w1_ref[D:2 * D, :] + b1_ref[...], 0.0)
    x = jnp.maximum(x @ w2_ref[...] + b2_ref[...], 0.0)
    score_ref[...] = x @ w3_ref[...] + b3_ref[...]


_BLK = 2048


def _tc_compute(h, ts2, tw, tb, mw, mb, wih, whh, rb, w1, b1, w2, b2, w3, b3):
    full = lambda shape: pl.BlockSpec(shape, lambda i: (0, 0))
    return pl.pallas_call(
        _tc_body,
        grid=(B // _BLK,),
        in_specs=[
            pl.BlockSpec((_BLK, D), lambda i: (i, 0)),
            pl.BlockSpec((_BLK, 1), lambda i: (i, 0)),
            full((1, D)), full((1, D)),
            full((2 * D, D)), full((1, D)),
            full((D, D)), full((D, D)), full((1, D)),
            full((2 * D, 64)), full((1, 64)),
            full((64, 16)), full((1, 16)),
            full((16, 1)), full((1, 1)),
        ],
        out_specs=[
            pl.BlockSpec((_BLK, D), lambda i: (i, 0)),
            pl.BlockSpec((_BLK, 1), lambda i: (i, 0)),
        ],
        out_shape=[
            jax.ShapeDtypeStruct((B, D), jnp.float32),
            jax.ShapeDtypeStruct((B, 1), jnp.float32),
        ],
    )(h, ts2, tw, tb, mw, mb, wih, whh, rb, w1, b1, w2, b2, w3, b3)


@functools.partial(
    pl.kernel,
    mesh=_mesh,
    out_type=(
        jax.ShapeDtypeStruct((D, N), jnp.float32),
        jax.ShapeDtypeStruct((D, NTAIL), jnp.float32),
    ),
    compiler_params=_params,
    scratch_types=[
        pltpu.VMEM((IDS_BLK,), jnp.int32),
        pltpu.VMEM((OWN_SZ,), jnp.int32),
        pltpu.VMEM((STG,), jnp.float32),
        pltpu.VMEM((D, CHN), jnp.float32),
        pltpu.VMEM((D, CHN), jnp.float32),
        pltpu.VMEM((D, 128), jnp.float32),
        pltpu.VMEM((D, NTAIL), jnp.float32),
        pltpu.SemaphoreType.DMA,
        pltpu.SemaphoreType.DMA,
        pltpu.SemaphoreType.DMA,
        pltpu.SemaphoreType.DMA,
    ],
)
def _sc_scatter_copy(mem_hbm, ids_hbm, newh_hbm, tail_hbm, out_hbm,
                     otail_hbm, idsb_v, own_v, stg_v, cb0_v, cb1_v, rb_v,
                     tb_v, sem_a, sem_b, sem_o, sem_w):
    wid = lax.axis_index("s") * NC + lax.axis_index("c")
    lo, span = _wrange(wid)
    iota = lax.iota(jnp.int32, L)
    span_t = jnp.where(wid == NW - 1, span + NTAIL, span)

    neg1 = jnp.full((L,), -1, jnp.int32)

    def init_body(i, _):
        own_v[pl.ds(i * L, L)] = neg1
        return 0

    lax.fori_loop(0, OWN_SZ // L, init_body, 0)

    def p1_blk(b, _):
        pltpu.sync_copy(ids_hbm.at[pl.ds(b * IDS_BLK, IDS_BLK)], idsb_v)

        def p1_vec(i, _):
            ids = idsb_v[pl.ds(i * L, L)]
            pos = iota + (b * IDS_BLK + i * L)
            rel = ids - lo
            m = (rel >= 0) & (rel < span_t)
            idx = jnp.where(m, rel, 0)
            plsc.store_scatter(own_v, [idx], pos, mask=m)
            return 0

        lax.fori_loop(0, IDS_BLK // L, p1_vec, 0)
        return 0

    lax.fori_loop(0, B // IDS_BLK, p1_blk, 0)

    def patch(bufview, crel, csz):
        def pvec(v, _):
            own = own_v[pl.ds(crel + v * L, L)]
            m = own >= 0
            nm = jnp.sum(jnp.where(m, 1, 0))

            @pl.when(nm > 0)
            def _():
                for l in range(L):
                    p = own[l]

                    @pl.when(p >= 0)
                    def _():
                        pltpu.async_copy(
                            newh_hbm.at[pl.ds(p * D, D)],
                            stg_v.at[pl.ds(l * 2 * L, 2 * L)],
                            sem_w,
                        )

                def drain(_k, _x):
                    pltpu.make_async_copy(
                        newh_hbm.at[pl.ds(0, D)], stg_v.at[pl.ds(0, D)], sem_w
                    ).wait()
                    return 0

                lax.fori_loop(0, nm, drain, 0)

                for l in range(L):
                    p = own[l]

                    @pl.when(p >= 0)
                    def _():
                        col = jnp.full((L,), v * L + l, jnp.int32)
                        v0 = stg_v[pl.ds(l * 2 * L, L)]
                        v1 = stg_v[pl.ds(l * 2 * L + L, L)]
                        plsc.store_scatter(bufview, [iota, col], v0)
                        plsc.store_scatter(bufview, [iota + L, col], v1)

            return 0

        lax.fori_loop(0, csz // L, pvec, 0)

    nch = span // CHN
    nrem = (span - nch * CHN) // 128

    def cin(c, buf, sem):
        return pltpu.make_async_copy(
            mem_hbm.at[:, pl.ds(lo + c * CHN, CHN)], buf, sem)

    def cout(c, buf):
        return pltpu.make_async_copy(
            buf, out_hbm.at[:, pl.ds(lo + c * CHN, CHN)], sem_o)

    @pl.when(nch > 0)
    def _():
        cin(0, cb0_v, sem_a).start()

        def body(c, _):
            even = c % 2 == 0

            @pl.when(even)
            def _():
                @pl.when(c >= 1)
                def _():
                    cout(c - 1, cb1_v).wait()

                @pl.when(c + 1 < nch)
                def _():
                    cin(c + 1, cb1_v, sem_b).start()

                cin(c, cb0_v, sem_a).wait()
                patch(cb0_v, c * CHN, CHN)
                cout(c, cb0_v).start()

            @pl.when(~even)
            def _():
                cout(c - 1, cb0_v).wait()

                @pl.when(c + 1 < nch)
                def _():
                    cin(c + 1, cb0_v, sem_a).start()

                cin(c, cb1_v, sem_b).wait()
                patch(cb1_v, c * CHN, CHN)
                cout(c, cb1_v).start()

            return 0

        lax.fori_loop(0, nch, body, 0)

        def last_wait():
            pass

        @pl.when(nch % 2 == 1)
        def _():
            cout(nch - 1, cb0_v).wait()

        @pl.when(nch % 2 == 0)
        def _():
            cout(nch - 1, cb1_v).wait()

    def rem_body(r, _):
        off = nch * CHN + r * 128
        pltpu.async_copy(
            mem_hbm.at[:, pl.ds(lo + off, 128)], rb_v, sem_a).wait()
        patch(rb_v, off, 128)
        pltpu.async_copy(
            rb_v, out_hbm.at[:, pl.ds(lo + off, 128)], sem_o).wait()
        return 0

    lax.fori_loop(0, nrem, rem_body, 0)

    @pl.when(wid == NW - 1)
    def _():
        pltpu.sync_copy(tail_hbm, tb_v)
        patch(tb_v, span, NTAIL)
        pltpu.sync_copy(tb_v, otail_hbm)


def kernel(memory, node_ids, timestamps, time_w, time_b, msg_W, msg_b,
           rnn_Wih, rnn_Whh, rnn_b, dec_W1, dec_b1, dec_W2, dec_b2,
           dec_W3, dec_b3):
    ids = node_ids.astype(jnp.int32)
    mem_t = memory.T
    mem_tail = lax.slice(mem_t, (0, TAIL), (D, N))
    h_flat = _sc_gather(mem_t, ids, mem_tail)
    h = h_flat.reshape(B, D)
    newh, score = _tc_compute(
        h, timestamps.reshape(B, 1), time_w.reshape(1, D),
        time_b.reshape(1, D), msg_W, msg_b.reshape(1, D), rnn_Wih,
        rnn_Whh, rnn_b.reshape(1, D), dec_W1, dec_b1.reshape(1, 64),
        dec_W2, dec_b2.reshape(1, 16), dec_W3, dec_b3.reshape(1, 1))
    newh_flat = newh.reshape(B * D)
    out_main, out_tail = _sc_scatter_copy(mem_t, ids, newh_flat, mem_tail)
    out_t = lax.dynamic_update_slice(out_main, out_tail, (0, TAIL))
    return (out_t.T, score)


# speedup vs baseline: 5.6282x; 1.8037x over previous
import functools

import jax
import jax.numpy as jnp
from jax import lax
from jax.experimental import pallas as pl
from jax.experimental.pallas import tpu as pltpu
from jax.experimental.pallas import tpu_sc as plsc

N = 1000000
D = 32
B = 16384

NC = 2
NS = 16
NW = NC * NS
L = 16

TCOLS = N // 128
TAIL = TCOLS * 128
NTAIL = N - TAIL
TC_BASE = TCOLS // NW
TC_REM = TCOLS % NW
CHT = 5
CHN = CHT * 128
OWN_SZ = (TC_BASE + 2) * 128
IDS_BLK = 2048
STG = 512
BIG = 1 << 30

_mesh = plsc.VectorSubcoreMesh(
    core_axis_name="c", subcore_axis_name="s", num_cores=NC, num_subcores=NS)
_params = pltpu.CompilerParams(needs_layout_passes=False)


def _popcnt(m):
    return plsc.all_reduce_population_count(m)[0]


def _wrange(wid):
    tc0 = wid * TC_BASE + jnp.minimum(wid, TC_REM)
    tc1 = (wid + 1) * TC_BASE + jnp.minimum(wid + 1, TC_REM)
    return tc0 * 128, (tc1 - tc0) * 128


def _build_owner(ids_hbm, idsb_v, own_v, lo, span_t):
    iota = lax.iota(jnp.int32, L)
    neg1 = jnp.full((L,), -1, jnp.int32)

    def init_body(i, _):
        own_v[pl.ds(i * L, L)] = neg1
        return 0

    lax.fori_loop(0, OWN_SZ // L, init_body, 0)

    def blk(b, _):
        pltpu.sync_copy(ids_hbm.at[pl.ds(b * IDS_BLK, IDS_BLK)], idsb_v)

        def vec(i, _):
            ids = idsb_v[pl.ds(i * L, L)]
            pos = iota + (b * IDS_BLK + i * L)
            rel = ids - lo
            m = (rel >= 0) & (rel < span_t)
            idx = jnp.where(m, rel, 0)
            plsc.store_scatter(own_v, [idx], pos, mask=m)
            return 0

        lax.fori_loop(0, IDS_BLK // L, vec, 0)
        return 0

    lax.fori_loop(0, B // IDS_BLK, blk, 0)


def _compact_winners(own_v, wrel_v, wpos_v):
    iota = lax.iota(jnp.int32, L)

    def vec(v, cnt):
        own = own_v[pl.ds(v * L, L)]
        m = own >= 0
        nm = _popcnt(m)

        @pl.when(nm > 0)
        def _():
            plsc.store_compressed(wrel_v.at[pl.ds(cnt, L)], iota + v * L,
                                  mask=m)
            plsc.store_compressed(wpos_v.at[pl.ds(cnt, L)], own, mask=m)

        return cnt + nm

    cnt = lax.fori_loop(0, OWN_SZ // L, vec, jnp.int32(0))
    wrel_v[pl.ds(cnt, L)] = jnp.full((L,), BIG, jnp.int32)
    return cnt


@functools.partial(
    pl.kernel,
    mesh=_mesh,
    out_type=jax.ShapeDtypeStruct((B * D,), jnp.float32),
    compiler_params=_params,
    scratch_types=[
        pltpu.VMEM((IDS_BLK,), jnp.int32),
        pltpu.VMEM((OWN_SZ,), jnp.int32),
        pltpu.VMEM((B + L,), jnp.int32),
        pltpu.VMEM((B + L,), jnp.int32),
        pltpu.VMEM((STG,), jnp.float32),
        pltpu.VMEM((D, CHN), jnp.float32),
        pltpu.VMEM((D, CHN), jnp.float32),
        pltpu.VMEM((D, 128), jnp.float32),
        pltpu.VMEM((D, NTAIL), jnp.float32),
        pltpu.SemaphoreType.DMA,
        pltpu.SemaphoreType.DMA,
        pltpu.SemaphoreType.DMA,
    ],
)
def _sc_gather(mem_hbm, ids_hbm, tail_hbm, h_hbm, idsb_v, own_v, wrel_v,
               wpos_v, stg_v, cb0_v, cb1_v, rb_v, tb_v, sem_a, sem_b, sem_h):
    wid = lax.axis_index("s") * NC + lax.axis_index("c")
    lo, span = _wrange(wid)
    iota = lax.iota(jnp.int32, L)
    span_t = jnp.where(wid == NW - 1, span + NTAIL, span)

    _build_owner(ids_hbm, idsb_v, own_v, lo, span_t)
    _compact_winners(own_v, wrel_v, wpos_v)

    def extract(bufview, crel, csz, cur):
        chi = crel + csz

        def cond(cur):
            return wrel_v[pl.ds(cur, L)][0] < chi

        def wbody(cur):
            relv = wrel_v[pl.ds(cur, L)]
            posv = wpos_v[pl.ds(cur, L)]
            k = _popcnt(relv < chi)
            for l in range(L):
                @pl.when(l < k)
                def _():
                    col = jnp.full((L,), relv[l] - crel, jnp.int32)
                    v0 = plsc.load_gather(bufview, [iota, col])
                    v1 = plsc.load_gather(bufview, [iota + L, col])
                    stg_v[pl.ds(l * 2 * L, L)] = v0
                    stg_v[pl.ds(l * 2 * L + L, L)] = v1
                    pltpu.async_copy(
                        stg_v.at[pl.ds(l * 2 * L, 2 * L)],
                        h_hbm.at[pl.ds(posv[l] * D, D)],
                        sem_h,
                    )

            def drain(_k, _x):
                pltpu.make_async_copy(
                    stg_v.at[pl.ds(0, D)], h_hbm.at[pl.ds(0, D)], sem_h
                ).wait()
                return 0

            lax.fori_loop(0, k, drain, 0)
            return cur + k

        return lax.while_loop(cond, wbody, cur)

    nch = span // CHN
    nrem = (span - nch * CHN) // 128

    def cin(c, buf, sem):
        return pltpu.make_async_copy(
            mem_hbm.at[:, pl.ds(lo + c * CHN, CHN)], buf, sem)

    def main_chunks(cur):
        cin(0, cb0_v, sem_a).start()

        def body(c, cur):
            def even_fn():
                @pl.when(c + 1 < nch)
                def _():
                    cin(c + 1, cb1_v, sem_b).start()

                cin(c, cb0_v, sem_a).wait()
                return extract(cb0_v, c * CHN, CHN, cur)

            def odd_fn():
                @pl.when(c + 1 < nch)
                def _():
                    cin(c + 1, cb0_v, sem_a).start()

                cin(c, cb1_v, sem_b).wait()
                return extract(cb1_v, c * CHN, CHN, cur)

            return lax.cond(c % 2 == 0, even_fn, odd_fn)

        return lax.fori_loop(0, nch, body, cur)

    cur = lax.cond(nch > 0, lambda: main_chunks(jnp.int32(0)),
                   lambda: jnp.int32(0))

    def rem_body(r, cur):
        off = nch * CHN + r * 128
        pltpu.async_copy(
            mem_hbm.at[:, pl.ds(lo + off, 128)], rb_v, sem_a).wait()
        return extract(rb_v, off, 128, cur)

    cur = lax.fori_loop(0, nrem, rem_body, cur)

    @pl.when(wid == NW - 1)
    def _():
        pltpu.sync_copy(tail_hbm, tb_v)
        extract(tb_v, span, NTAIL, cur)

    def dup_blk(b, _):
        pltpu.sync_copy(ids_hbm.at[pl.ds(b * IDS_BLK, IDS_BLK)], idsb_v)

        def dup_vec(i, _):
            ids = idsb_v[pl.ds(i * L, L)]
            pos = iota + (b * IDS_BLK + i * L)
            rel = ids - lo
            m = (rel >= 0) & (rel < span_t)
            idx = jnp.where(m, rel, 0)
            own = plsc.load_gather(own_v, [idx], mask=m)
            dup = m & (own != pos)
            nd = _popcnt(dup)

            dupi = jnp.where(dup, 1, 0)

            @pl.when(nd > 0)
            def _():
                for l in range(L):
                    sel = dupi[l] == 1

                    @pl.when(sel)
                    def _():
                        pltpu.async_copy(
                            h_hbm.at[pl.ds(own[l] * D, D)],
                            stg_v.at[pl.ds(l * 2 * L, 2 * L)],
                            sem_h,
                        )

                def drain(_k, _x):
                    pltpu.make_async_copy(
                        h_hbm.at[pl.ds(0, D)], stg_v.at[pl.ds(0, D)], sem_h
                    ).wait()
                    return 0

                lax.fori_loop(0, nd, drain, 0)

                for l in range(L):
                    sel = dupi[l] == 1

                    @pl.when(sel)
                    def _():
                        pltpu.async_copy(
                            stg_v.at[pl.ds(l * 2 * L, 2 * L)],
                            h_hbm.at[pl.ds(pos[l] * D, D)],
                            sem_h,
                        )

                lax.fori_loop(0, nd, drain, 0)

            return 0

        lax.fori_loop(0, IDS_BLK // L, dup_vec, 0)
        return 0

    lax.fori_loop(0, B // IDS_BLK, dup_blk, 0)


def _tc_body(h_ref, ts_ref, tw_ref, tb_ref, mw_ref, mb_ref, wih_ref, whh_ref,
             rb_ref, w1_ref, b1_ref, w2_ref, b2_ref, w3_ref, b3_ref,
             newh_ref, score_ref):
    h = h_ref[...]
    te = jnp.cos(ts_ref[...] * tw_ref[...] + tb_ref[...])
    msg = jnp.maximum(
        h @ mw_ref[0:D, :] + te @ mw_ref[D:2 * D, :] + mb_ref[...], 0.0)
    nh = jnp.tanh(msg @ wih_ref[...] + h @ whh_ref[...] + rb_ref[...])
    newh_ref[...] = nh
    x = jnp.maximum(
        h @ w1_ref[0:D, :] + nh @ w1_ref[D:2 * D, :] + b1_ref[...], 0.0)
    x = jnp.maximum(x @ w2_ref[...] + b2_ref[...], 0.0)
    score_ref[...] = x @ w3_ref[...] + b3_ref[...]


_BLK = 2048


def _tc_compute(h, ts2, tw, tb, mw, mb, wih, whh, rb, w1, b1, w2, b2, w3, b3):
    full = lambda shape: pl.BlockSpec(shape, lambda i: (0, 0))
    return pl.pallas_call(
        _tc_body,
        grid=(B // _BLK,),
        in_specs=[
            pl.BlockSpec((_BLK, D), lambda i: (i, 0)),
            pl.BlockSpec((_BLK, 1), lambda i: (i, 0)),
            full((1, D)), full((1, D)),
            full((2 * D, D)), full((1, D)),
            full((D, D)), full((D, D)), full((1, D)),
            full((2 * D, 64)), full((1, 64)),
            full((64, 16)), full((1, 16)),
            full((16, 1)), full((1, 1)),
        ],
        out_specs=[
            pl.BlockSpec((_BLK, D), lambda i: (i, 0)),
            pl.BlockSpec((_BLK, 1), lambda i: (i, 0)),
        ],
        out_shape=[
            jax.ShapeDtypeStruct((B, D), jnp.float32),
            jax.ShapeDtypeStruct((B, 1), jnp.float32),
        ],
    )(h, ts2, tw, tb, mw, mb, wih, whh, rb, w1, b1, w2, b2, w3, b3)


@functools.partial(
    pl.kernel,
    mesh=_mesh,
    out_type=(
        jax.ShapeDtypeStruct((D, N), jnp.float32),
        jax.ShapeDtypeStruct((D, NTAIL), jnp.float32),
    ),
    compiler_params=_params,
    scratch_types=[
        pltpu.VMEM((IDS_BLK,), jnp.int32),
        pltpu.VMEM((OWN_SZ,), jnp.int32),
        pltpu.VMEM((B + L,), jnp.int32),
        pltpu.VMEM((B + L,), jnp.int32),
        pltpu.VMEM((STG,), jnp.float32),
        pltpu.VMEM((D, CHN), jnp.float32),
        pltpu.VMEM((D, CHN), jnp.float32),
        pltpu.VMEM((D, 128), jnp.float32),
        pltpu.VMEM((D, NTAIL), jnp.float32),
        pltpu.SemaphoreType.DMA,
        pltpu.SemaphoreType.DMA,
        pltpu.SemaphoreType.DMA,
        pltpu.SemaphoreType.DMA,
    ],
)
def _sc_scatter_copy(mem_hbm, ids_hbm, newh_hbm, tail_hbm, out_hbm,
                     otail_hbm, idsb_v, own_v, wrel_v, wpos_v, stg_v,
                     cb0_v, cb1_v, rb_v, tb_v, sem_a, sem_b, sem_o, sem_w):
    wid = lax.axis_index("s") * NC + lax.axis_index("c")
    lo, span = _wrange(wid)
    iota = lax.iota(jnp.int32, L)
    span_t = jnp.where(wid == NW - 1, span + NTAIL, span)

    _build_owner(ids_hbm, idsb_v, own_v, lo, span_t)
    _compact_winners(own_v, wrel_v, wpos_v)

    def patch(bufview, crel, csz, cur):
        chi = crel + csz

        def cond(cur):
            return wrel_v[pl.ds(cur, L)][0] < chi

        def wbody(cur):
            relv = wrel_v[pl.ds(cur, L)]
            posv = wpos_v[pl.ds(cur, L)]
            k = _popcnt(relv < chi)
            for l in range(L):
                @pl.when(l < k)
                def _():
                    pltpu.async_copy(
                        newh_hbm.at[pl.ds(posv[l] * D, D)],
                        stg_v.at[pl.ds(l * 2 * L, 2 * L)],
                        sem_w,
                    )

            def drain(_k, _x):
                pltpu.make_async_copy(
                    newh_hbm.at[pl.ds(0, D)], stg_v.at[pl.ds(0, D)], sem_w
                ).wait()
                return 0

            lax.fori_loop(0, k, drain, 0)

            for l in range(L):
                @pl.when(l < k)
                def _():
                    col = jnp.full((L,), relv[l] - crel, jnp.int32)
                    v0 = stg_v[pl.ds(l * 2 * L, L)]
                    v1 = stg_v[pl.ds(l * 2 * L + L, L)]
                    plsc.store_scatter(bufview, [iota, col], v0)
                    plsc.store_scatter(bufview, [iota + L, col], v1)

            return cur + k

        return lax.while_loop(cond, wbody, cur)

    nch = span // CHN
    nrem = (span - nch * CHN) // 128

    def cin(c, buf, sem):
        return pltpu.make_async_copy(
            mem_hbm.at[:, pl.ds(lo + c * CHN, CHN)], buf, sem)

    def cout(c, buf):
        return pltpu.make_async_copy(
            buf, out_hbm.at[:, pl.ds(lo + c * CHN, CHN)], sem_o)

    def main_chunks(cur):
        cin(0, cb0_v, sem_a).start()

        def body(c, cur):
            def even_fn():
                @pl.when(c >= 1)
                def _():
                    cout(c - 1, cb1_v).wait()

                @pl.when(c + 1 < nch)
                def _():
                    cin(c + 1, cb1_v, sem_b).start()

                cin(c, cb0_v, sem_a).wait()
                ncur = patch(cb0_v, c * CHN, CHN, cur)
                cout(c, cb0_v).start()
                return ncur

            def odd_fn():
                cout(c - 1, cb0_v).wait()

                @pl.when(c + 1 < nch)
                def _():
                    cin(c + 1, cb0_v, sem_a).start()

                cin(c, cb1_v, sem_b).wait()
                ncur = patch(cb1_v, c * CHN, CHN, cur)
                cout(c, cb1_v).start()
                return ncur

            return lax.cond(c % 2 == 0, even_fn, odd_fn)

        cur = lax.fori_loop(0, nch, body, cur)

        @pl.when(nch % 2 == 1)
        def _():
            cout(nch - 1, cb0_v).wait()

        @pl.when(nch % 2 == 0)
        def _():
            cout(nch - 1, cb1_v).wait()

        return cur

    cur = lax.cond(nch > 0, lambda: main_chunks(jnp.int32(0)),
                   lambda: jnp.int32(0))

    def rem_body(r, cur):
        off = nch * CHN + r * 128
        pltpu.async_copy(
            mem_hbm.at[:, pl.ds(lo + off, 128)], rb_v, sem_a).wait()
        cur = patch(rb_v, off, 128, cur)
        pltpu.async_copy(
            rb_v, out_hbm.at[:, pl.ds(lo + off, 128)], sem_o).wait()
        return cur

    cur = lax.fori_loop(0, nrem, rem_body, cur)

    @pl.when(wid == NW - 1)
    def _():
        pltpu.sync_copy(tail_hbm, tb_v)
        patch(tb_v, span, NTAIL, cur)
        pltpu.sync_copy(tb_v, otail_hbm)


def kernel(memory, node_ids, timestamps, time_w, time_b, msg_W, msg_b,
           rnn_Wih, rnn_Whh, rnn_b, dec_W1, dec_b1, dec_W2, dec_b2,
           dec_W3, dec_b3):
    ids = node_ids.astype(jnp.int32)
    mem_t = memory.T
    mem_tail = lax.slice(mem_t, (0, TAIL), (D, N))
    h_flat = _sc_gather(mem_t, ids, mem_tail)
    h = h_flat.reshape(B, D)
    newh, score = _tc_compute(
        h, timestamps.reshape(B, 1), time_w.reshape(1, D),
        time_b.reshape(1, D), msg_W, msg_b.reshape(1, D), rnn_Wih,
        rnn_Whh, rnn_b.reshape(1, D), dec_W1, dec_b1.reshape(1, 64),
        dec_W2, dec_b2.reshape(1, 16), dec_W3, dec_b3.reshape(1, 1))
    newh_flat = newh.reshape(B * D)
    out_main, out_tail = _sc_scatter_copy(mem_t, ids, newh_flat, mem_tail)
    out_t = lax.dynamic_update_slice(out_main, out_tail, (0, TAIL))
    return (out_t.T, score)


# speedup vs baseline: 6.4781x; 1.1510x over previous
import functools

import jax
import jax.numpy as jnp
from jax import lax
from jax.experimental import pallas as pl
from jax.experimental.pallas import tpu as pltpu
from jax.experimental.pallas import tpu_sc as plsc

N = 1000000
D = 32
B = 16384

NC = 2
NS = 16
NW = NC * NS
L = 16

TCOLS = N // 128
TAIL = TCOLS * 128
NTAIL = N - TAIL
TC_BASE = TCOLS // NW
TC_REM = TCOLS % NW
CHT = 5
CHN = CHT * 128
OWN_SZ = (TC_BASE + 2) * 128
IDS_BLK = 2048
STG = 512
BIG = 1 << 30

_mesh = plsc.VectorSubcoreMesh(
    core_axis_name="c", subcore_axis_name="s", num_cores=NC, num_subcores=NS)
_params = pltpu.CompilerParams(needs_layout_passes=False)


def _popcnt(m):
    return plsc.all_reduce_population_count(m)[0]


def _wrange(wid):
    tc0 = wid * TC_BASE + jnp.minimum(wid, TC_REM)
    tc1 = (wid + 1) * TC_BASE + jnp.minimum(wid + 1, TC_REM)
    return tc0 * 128, (tc1 - tc0) * 128


def _build_owner(ids_hbm, idsb_v, own_v, lo, span_t):
    iota = lax.iota(jnp.int32, L)
    neg1 = jnp.full((L,), -1, jnp.int32)

    def init_body(i, _):
        own_v[pl.ds(i * L, L)] = neg1
        return 0

    lax.fori_loop(0, OWN_SZ // L, init_body, 0)

    def blk(b, _):
        pltpu.sync_copy(ids_hbm.at[pl.ds(b * IDS_BLK, IDS_BLK)], idsb_v)

        def vec(i, _):
            ids = idsb_v[pl.ds(i * L, L)]
            pos = iota + (b * IDS_BLK + i * L)
            rel = ids - lo
            m = (rel >= 0) & (rel < span_t)
            idx = jnp.where(m, rel, 0)
            plsc.store_scatter(own_v, [idx], pos, mask=m)
            return 0

        lax.fori_loop(0, IDS_BLK // L, vec, 0)
        return 0

    lax.fori_loop(0, B // IDS_BLK, blk, 0)


def _compact_winners(own_v, wrel_v, wpos_v):
    iota = lax.iota(jnp.int32, L)

    def vec(v4, cnt):
        for u in range(4):
            v = v4 * 4 + u
            own = own_v[pl.ds(v * L, L)]
            m = own >= 0
            nm = _popcnt(m)

            @pl.when(nm > 0)
            def _():
                plsc.store_compressed(wrel_v.at[pl.ds(cnt, L)], iota + v * L,
                                      mask=m)
                plsc.store_compressed(wpos_v.at[pl.ds(cnt, L)], own, mask=m)

            cnt = cnt + nm
        return cnt

    cnt = lax.fori_loop(0, OWN_SZ // (4 * L), vec, jnp.int32(0))
    wrel_v[pl.ds(cnt, L)] = jnp.full((L,), BIG, jnp.int32)
    return cnt


@functools.partial(
    pl.kernel,
    mesh=_mesh,
    out_type=(
        jax.ShapeDtypeStruct((B * D,), jnp.float32),
        jax.ShapeDtypeStruct((NW * (B + L),), jnp.int32),
        jax.ShapeDtypeStruct((NW * (B + L),), jnp.int32),
    ),
    compiler_params=_params,
    scratch_types=[
        pltpu.VMEM((IDS_BLK,), jnp.int32),
        pltpu.VMEM((OWN_SZ,), jnp.int32),
        pltpu.VMEM((B + L,), jnp.int32),
        pltpu.VMEM((B + L,), jnp.int32),
        pltpu.VMEM((STG,), jnp.float32),
        pltpu.VMEM((D, CHN), jnp.float32),
        pltpu.VMEM((D, CHN), jnp.float32),
        pltpu.VMEM((D, 128), jnp.float32),
        pltpu.VMEM((D, NTAIL), jnp.float32),
        pltpu.SemaphoreType.DMA,
        pltpu.SemaphoreType.DMA,
        pltpu.SemaphoreType.DMA,
    ],
)
def _sc_gather(mem_hbm, ids_hbm, tail_hbm, h_hbm, wrelx_hbm, wposx_hbm,
               idsb_v, own_v, wrel_v, wpos_v, stg_v, cb0_v, cb1_v, rb_v,
               tb_v, sem_a, sem_b, sem_h):
    wid = lax.axis_index("s") * NC + lax.axis_index("c")
    lo, span = _wrange(wid)
    iota = lax.iota(jnp.int32, L)
    span_t = jnp.where(wid == NW - 1, span + NTAIL, span)

    _build_owner(ids_hbm, idsb_v, own_v, lo, span_t)
    _compact_winners(own_v, wrel_v, wpos_v)
    pltpu.sync_copy(wrel_v, wrelx_hbm.at[pl.ds(wid * (B + L), B + L)])
    pltpu.sync_copy(wpos_v, wposx_hbm.at[pl.ds(wid * (B + L), B + L)])

    def extract(bufview, crel, csz, cur):
        chi = crel + csz

        def cond(cur):
            return wrel_v[pl.ds(cur, L)][0] < chi

        def wbody(cur):
            relv = wrel_v[pl.ds(cur, L)]
            posv = wpos_v[pl.ds(cur, L)]
            k = _popcnt(relv < chi)
            for l in range(L):
                @pl.when(l < k)
                def _():
                    col = jnp.full((L,), relv[l] - crel, jnp.int32)
                    v0 = plsc.load_gather(bufview, [iota, col])
                    v1 = plsc.load_gather(bufview, [iota + L, col])
                    stg_v[pl.ds(l * 2 * L, L)] = v0
                    stg_v[pl.ds(l * 2 * L + L, L)] = v1
                    pltpu.async_copy(
                        stg_v.at[pl.ds(l * 2 * L, 2 * L)],
                        h_hbm.at[pl.ds(posv[l] * D, D)],
                        sem_h,
                    )

            def drain(_k, _x):
                pltpu.make_async_copy(
                    stg_v.at[pl.ds(0, D)], h_hbm.at[pl.ds(0, D)], sem_h
                ).wait()
                return 0

            lax.fori_loop(0, k, drain, 0)
            return cur + k

        return lax.while_loop(cond, wbody, cur)

    nch = span // CHN
    nrem = (span - nch * CHN) // 128

    def cin(c, buf, sem):
        return pltpu.make_async_copy(
            mem_hbm.at[:, pl.ds(lo + c * CHN, CHN)], buf, sem)

    def main_chunks(cur):
        cin(0, cb0_v, sem_a).start()

        def body(c, cur):
            def even_fn():
                @pl.when(c + 1 < nch)
                def _():
                    cin(c + 1, cb1_v, sem_b).start()

                cin(c, cb0_v, sem_a).wait()
                return extract(cb0_v, c * CHN, CHN, cur)

            def odd_fn():
                @pl.when(c + 1 < nch)
                def _():
                    cin(c + 1, cb0_v, sem_a).start()

                cin(c, cb1_v, sem_b).wait()
                return extract(cb1_v, c * CHN, CHN, cur)

            return lax.cond(c % 2 == 0, even_fn, odd_fn)

        return lax.fori_loop(0, nch, body, cur)

    cur = lax.cond(nch > 0, lambda: main_chunks(jnp.int32(0)),
                   lambda: jnp.int32(0))

    def rem_body(r, cur):
        off = nch * CHN + r * 128
        pltpu.async_copy(
            mem_hbm.at[:, pl.ds(lo + off, 128)], rb_v, sem_a).wait()
        return extract(rb_v, off, 128, cur)

    cur = lax.fori_loop(0, nrem, rem_body, cur)

    @pl.when(wid == NW - 1)
    def _():
        pltpu.sync_copy(tail_hbm, tb_v)
        extract(tb_v, span, NTAIL, cur)

    def dup_blk(b, _):
        pltpu.sync_copy(ids_hbm.at[pl.ds(b * IDS_BLK, IDS_BLK)], idsb_v)

        def dup_vec(i, _):
            ids = idsb_v[pl.ds(i * L, L)]
            pos = iota + (b * IDS_BLK + i * L)
            rel = ids - lo
            m = (rel >= 0) & (rel < span_t)
            idx = jnp.where(m, rel, 0)
            own = plsc.load_gather(own_v, [idx], mask=m)
            dup = m & (own != pos)
            nd = _popcnt(dup)

            dupi = jnp.where(dup, 1, 0)

            @pl.when(nd > 0)
            def _():
                for l in range(L):
                    sel = dupi[l] == 1

                    @pl.when(sel)
                    def _():
                        pltpu.async_copy(
                            h_hbm.at[pl.ds(own[l] * D, D)],
                            stg_v.at[pl.ds(l * 2 * L, 2 * L)],
                            sem_h,
                        )

                def drain(_k, _x):
                    pltpu.make_async_copy(
                        h_hbm.at[pl.ds(0, D)], stg_v.at[pl.ds(0, D)], sem_h
                    ).wait()
                    return 0

                lax.fori_loop(0, nd, drain, 0)

                for l in range(L):
                    sel = dupi[l] == 1

                    @pl.when(sel)
                    def _():
                        pltpu.async_copy(
                            stg_v.at[pl.ds(l * 2 * L, 2 * L)],
                            h_hbm.at[pl.ds(pos[l] * D, D)],
                            sem_h,
                        )

                lax.fori_loop(0, nd, drain, 0)

            return 0

        lax.fori_loop(0, IDS_BLK // L, dup_vec, 0)
        return 0

    lax.fori_loop(0, B // IDS_BLK, dup_blk, 0)


def _tc_body(h_ref, ts_ref, tw_ref, tb_ref, mw_ref, mb_ref, wih_ref, whh_ref,
             rb_ref, w1_ref, b1_ref, w2_ref, b2_ref, w3_ref, b3_ref,
             newh_ref, score_ref):
    h = h_ref[...]
    te = jnp.cos(ts_ref[...] * tw_ref[...] + tb_ref[...])
    msg = jnp.maximum(
        h @ mw_ref[0:D, :] + te @ mw_ref[D:2 * D, :] + mb_ref[...], 0.0)
    nh = jnp.tanh(msg @ wih_ref[...] + h @ whh_ref[...] + rb_ref[...])
    newh_ref[...] = nh
    x = jnp.maximum(
        h @ w1_ref[0:D, :] + nh @ w1_ref[D:2 * D, :] + b1_ref[...], 0.0)
    x = jnp.maximum(x @ w2_ref[...] + b2_ref[...], 0.0)
    score_ref[...] = x @ w3_ref[...] + b3_ref[...]


_BLK = 2048


def _tc_compute(h, ts2, tw, tb, mw, mb, wih, whh, rb, w1, b1, w2, b2, w3, b3):
    full = lambda shape: pl.BlockSpec(shape, lambda i: (0, 0))
    return pl.pallas_call(
        _tc_body,
        grid=(B // _BLK,),
        in_specs=[
            pl.BlockSpec((_BLK, D), lambda i: (i, 0)),
            pl.BlockSpec((_BLK, 1), lambda i: (i, 0)),
            full((1, D)), full((1, D)),
            full((2 * D, D)), full((1, D)),
            full((D, D)), full((D, D)), full((1, D)),
            full((2 * D, 64)), full((1, 64)),
            full((64, 16)), full((1, 16)),
            full((16, 1)), full((1, 1)),
        ],
        out_specs=[
            pl.BlockSpec((_BLK, D), lambda i: (i, 0)),
            pl.BlockSpec((_BLK, 1), lambda i: (i, 0)),
        ],
        out_shape=[
            jax.ShapeDtypeStruct((B, D), jnp.float32),
            jax.ShapeDtypeStruct((B, 1), jnp.float32),
        ],
    )(h, ts2, tw, tb, mw, mb, wih, whh, rb, w1, b1, w2, b2, w3, b3)


@functools.partial(
    pl.kernel,
    mesh=_mesh,
    out_type=(
        jax.ShapeDtypeStruct((D, N), jnp.float32),
        jax.ShapeDtypeStruct((D, NTAIL), jnp.float32),
    ),
    compiler_params=_params,
    scratch_types=[
        pltpu.VMEM((B + L,), jnp.int32),
        pltpu.VMEM((B + L,), jnp.int32),
        pltpu.VMEM((STG,), jnp.float32),
        pltpu.VMEM((D, CHN), jnp.float32),
        pltpu.VMEM((D, CHN), jnp.float32),
        pltpu.VMEM((D, 128), jnp.float32),
        pltpu.VMEM((D, NTAIL), jnp.float32),
        pltpu.SemaphoreType.DMA,
        pltpu.SemaphoreType.DMA,
        pltpu.SemaphoreType.DMA,
        pltpu.SemaphoreType.DMA,
    ],
)
def _sc_scatter_copy(mem_hbm, newh_hbm, tail_hbm, wrelx_hbm, wposx_hbm,
                     out_hbm, otail_hbm, wrel_v, wpos_v, stg_v,
                     cb0_v, cb1_v, rb_v, tb_v, sem_a, sem_b, sem_o, sem_w):
    wid = lax.axis_index("s") * NC + lax.axis_index("c")
    lo, span = _wrange(wid)
    iota = lax.iota(jnp.int32, L)

    pltpu.sync_copy(wrelx_hbm.at[pl.ds(wid * (B + L), B + L)], wrel_v)
    pltpu.sync_copy(wposx_hbm.at[pl.ds(wid * (B + L), B + L)], wpos_v)

    def patch(bufview, crel, csz, cur):
        chi = crel + csz

        def cond(cur):
            return wrel_v[pl.ds(cur, L)][0] < chi

        def wbody(cur):
            relv = wrel_v[pl.ds(cur, L)]
            posv = wpos_v[pl.ds(cur, L)]
            k = _popcnt(relv < chi)
            for l in range(L):
                @pl.when(l < k)
                def _():
                    pltpu.async_copy(
                        newh_hbm.at[pl.ds(posv[l] * D, D)],
                        stg_v.at[pl.ds(l * 2 * L, 2 * L)],
                        sem_w,
                    )

            def drain(_k, _x):
                pltpu.make_async_copy(
                    newh_hbm.at[pl.ds(0, D)], stg_v.at[pl.ds(0, D)], sem_w
                ).wait()
                return 0

            lax.fori_loop(0, k, drain, 0)

            for l in range(L):
                @pl.when(l < k)
                def _():
                    col = jnp.full((L,), relv[l] - crel, jnp.int32)
                    v0 = stg_v[pl.ds(l * 2 * L, L)]
                    v1 = stg_v[pl.ds(l * 2 * L + L, L)]
                    plsc.store_scatter(bufview, [iota, col], v0)
                    plsc.store_scatter(bufview, [iota + L, col], v1)

            return cur + k

        return lax.while_loop(cond, wbody, cur)

    nch = span // CHN
    nrem = (span - nch * CHN) // 128

    def cin(c, buf, sem):
        return pltpu.make_async_copy(
            mem_hbm.at[:, pl.ds(lo + c * CHN, CHN)], buf, sem)

    def cout(c, buf):
        return pltpu.make_async_copy(
            buf, out_hbm.at[:, pl.ds(lo + c * CHN, CHN)], sem_o)

    def main_chunks(cur):
        cin(0, cb0_v, sem_a).start()

        def body(c, cur):
            def even_fn():
                @pl.when(c >= 1)
                def _():
                    cout(c - 1, cb1_v).wait()

                @pl.when(c + 1 < nch)
                def _():
                    cin(c + 1, cb1_v, sem_b).start()

                cin(c, cb0_v, sem_a).wait()
                ncur = patch(cb0_v, c * CHN, CHN, cur)
                cout(c, cb0_v).start()
                return ncur

            def odd_fn():
                cout(c - 1, cb0_v).wait()

                @pl.when(c + 1 < nch)
                def _():
                    cin(c + 1, cb0_v, sem_a).start()

                cin(c, cb1_v, sem_b).wait()
                ncur = patch(cb1_v, c * CHN, CHN, cur)
                cout(c, cb1_v).start()
                return ncur

            return lax.cond(c % 2 == 0, even_fn, odd_fn)

        cur = lax.fori_loop(0, nch, body, cur)

        @pl.when(nch % 2 == 1)
        def _():
            cout(nch - 1, cb0_v).wait()

        @pl.when(nch % 2 == 0)
        def _():
            cout(nch - 1, cb1_v).wait()

        return cur

    cur = lax.cond(nch > 0, lambda: main_chunks(jnp.int32(0)),
                   lambda: jnp.int32(0))

    def rem_body(r, cur):
        off = nch * CHN + r * 128
        pltpu.async_copy(
            mem_hbm.at[:, pl.ds(lo + off, 128)], rb_v, sem_a).wait()
        cur = patch(rb_v, off, 128, cur)
        pltpu.async_copy(
            rb_v, out_hbm.at[:, pl.ds(lo + off, 128)], sem_o).wait()
        return cur

    cur = lax.fori_loop(0, nrem, rem_body, cur)

    @pl.when(wid == NW - 1)
    def _():
        pltpu.sync_copy(tail_hbm, tb_v)
        patch(tb_v, span, NTAIL, cur)
        pltpu.sync_copy(tb_v, otail_hbm)


def kernel(memory, node_ids, timestamps, time_w, time_b, msg_W, msg_b,
           rnn_Wih, rnn_Whh, rnn_b, dec_W1, dec_b1, dec_W2, dec_b2,
           dec_W3, dec_b3):
    ids = node_ids.astype(jnp.int32)
    mem_t = memory.T
    mem_tail = lax.slice(mem_t, (0, TAIL), (D, N))
    h_flat, wrelx, wposx = _sc_gather(mem_t, ids, mem_tail)
    h = h_flat.reshape(B, D)
    newh, score = _tc_compute(
        h, timestamps.reshape(B, 1), time_w.reshape(1, D),
        time_b.reshape(1, D), msg_W, msg_b.reshape(1, D), rnn_Wih,
        rnn_Whh, rnn_b.reshape(1, D), dec_W1, dec_b1.reshape(1, 64),
        dec_W2, dec_b2.reshape(1, 16), dec_W3, dec_b3.reshape(1, 1))
    newh_flat = newh.reshape(B * D)
    out_main, out_tail = _sc_scatter_copy(mem_t, newh_flat, mem_tail,
                                          wrelx, wposx)
    out_t = lax.dynamic_update_slice(out_main, out_tail, (0, TAIL))
    return (out_t.T, score)


# speedup vs baseline: 7.0849x; 1.0937x over previous
import functools

import jax
import jax.numpy as jnp
from jax import lax
from jax.experimental import pallas as pl
from jax.experimental.pallas import tpu as pltpu
from jax.experimental.pallas import tpu_sc as plsc

N = 1000000
D = 32
B = 16384

NC = 2
NS = 16
NW = NC * NS
L = 16

TCOLS = N // 128
TAIL = TCOLS * 128
NTAIL = N - TAIL
TC_BASE = TCOLS // NW
TC_REM = TCOLS % NW
CHT = 5
CHN = CHT * 128
CH1 = 4 * 128
OWN_SZ = (TC_BASE + 2) * 128
IDS_BLK = 2048
STG = 512
BIG = 1 << 30

_mesh = plsc.VectorSubcoreMesh(
    core_axis_name="c", subcore_axis_name="s", num_cores=NC, num_subcores=NS)
_params = pltpu.CompilerParams(needs_layout_passes=False)


def _popcnt(m):
    return plsc.all_reduce_population_count(m)[0]


def _wrange(wid):
    tc0 = wid * TC_BASE + jnp.minimum(wid, TC_REM)
    tc1 = (wid + 1) * TC_BASE + jnp.minimum(wid + 1, TC_REM)
    return tc0 * 128, (tc1 - tc0) * 128


def _build_owner(ids_hbm, idsb_v, own_v, lo, span_t):
    iota = lax.iota(jnp.int32, L)
    neg1 = jnp.full((L,), -1, jnp.int32)

    def init_body(i, _):
        own_v[pl.ds(i * L, L)] = neg1
        return 0

    lax.fori_loop(0, OWN_SZ // L, init_body, 0)

    def blk(b, _):
        pltpu.sync_copy(ids_hbm.at[pl.ds(b * IDS_BLK, IDS_BLK)], idsb_v)

        def vec(i, _):
            ids = idsb_v[pl.ds(i * L, L)]
            pos = iota + (b * IDS_BLK + i * L)
            rel = ids - lo
            m = (rel >= 0) & (rel < span_t)
            idx = jnp.where(m, rel, 0)
            plsc.store_scatter(own_v, [idx], pos, mask=m)
            return 0

        lax.fori_loop(0, IDS_BLK // L, vec, 0)
        return 0

    lax.fori_loop(0, B // IDS_BLK, blk, 0)


def _compact_winners(own_v, wrel_v, wpos_v):
    iota = lax.iota(jnp.int32, L)

    def vec(v4, cnt):
        for u in range(4):
            v = v4 * 4 + u
            own = own_v[pl.ds(v * L, L)]
            m = own >= 0
            plsc.store_compressed(wrel_v.at[pl.ds(cnt, L)], iota + v * L,
                                  mask=m)
            plsc.store_compressed(wpos_v.at[pl.ds(cnt, L)], own, mask=m)
            cnt = cnt + _popcnt(m)
        return cnt

    cnt = lax.fori_loop(0, OWN_SZ // (4 * L), vec, jnp.int32(0))
    wrel_v[pl.ds(cnt, L)] = jnp.full((L,), BIG, jnp.int32)
    return cnt


@functools.partial(
    pl.kernel,
    mesh=_mesh,
    out_type=(
        jax.ShapeDtypeStruct((B * D,), jnp.float32),
        jax.ShapeDtypeStruct((NW * (B + L),), jnp.int32),
        jax.ShapeDtypeStruct((NW * (B + L),), jnp.int32),
    ),
    compiler_params=_params,
    scratch_types=[
        pltpu.VMEM((IDS_BLK,), jnp.int32),
        pltpu.VMEM((OWN_SZ,), jnp.int32),
        pltpu.VMEM((B + L,), jnp.int32),
        pltpu.VMEM((B + L,), jnp.int32),
        pltpu.VMEM((STG,), jnp.float32),
        pltpu.VMEM((D, CH1), jnp.float32),
        pltpu.VMEM((D, CH1), jnp.float32),
        pltpu.VMEM((D, CH1), jnp.float32),
        pltpu.VMEM((D, 128), jnp.float32),
        pltpu.VMEM((D, NTAIL), jnp.float32),
        pltpu.SemaphoreType.DMA,
        pltpu.SemaphoreType.DMA,
        pltpu.SemaphoreType.DMA,
        pltpu.SemaphoreType.DMA,
    ],
)
def _sc_gather(mem_hbm, ids_hbm, tail_hbm, h_hbm, wrelx_hbm, wposx_hbm,
               idsb_v, own_v, wrel_v, wpos_v, stg_v, cb0_v, cb1_v, cb2_v,
               rb_v, tb_v, sem_a, sem_b, sem_c, sem_h):
    wid = lax.axis_index("s") * NC + lax.axis_index("c")
    lo, span = _wrange(wid)
    iota = lax.iota(jnp.int32, L)
    span_t = jnp.where(wid == NW - 1, span + NTAIL, span)

    _build_owner(ids_hbm, idsb_v, own_v, lo, span_t)
    _compact_winners(own_v, wrel_v, wpos_v)
    pltpu.sync_copy(wrel_v, wrelx_hbm.at[pl.ds(wid * (B + L), B + L)])
    pltpu.sync_copy(wpos_v, wposx_hbm.at[pl.ds(wid * (B + L), B + L)])

    def extract(bufview, crel, csz, cur):
        chi = crel + csz

        def cond(cur):
            return wrel_v[pl.ds(cur, L)][0] < chi

        def wbody(cur):
            relv = wrel_v[pl.ds(cur, L)]
            posv = wpos_v[pl.ds(cur, L)]
            k = _popcnt(relv < chi)
            for l in range(L):
                @pl.when(l < k)
                def _():
                    col = jnp.full((L,), relv[l] - crel, jnp.int32)
                    v0 = plsc.load_gather(bufview, [iota, col])
                    v1 = plsc.load_gather(bufview, [iota + L, col])
                    stg_v[pl.ds(l * 2 * L, L)] = v0
                    stg_v[pl.ds(l * 2 * L + L, L)] = v1
                    pltpu.async_copy(
                        stg_v.at[pl.ds(l * 2 * L, 2 * L)],
                        h_hbm.at[pl.ds(posv[l] * D, D)],
                        sem_h,
                    )

            def drain(_k, _x):
                pltpu.make_async_copy(
                    stg_v.at[pl.ds(0, D)], h_hbm.at[pl.ds(0, D)], sem_h
                ).wait()
                return 0

            lax.fori_loop(0, k, drain, 0)
            return cur + k

        return lax.while_loop(cond, wbody, cur)

    nch = span // CH1
    nrem = (span - nch * CH1) // 128
    bufs = (cb0_v, cb1_v, cb2_v)
    sems = (sem_a, sem_b, sem_c)

    def cin(c, buf, sem):
        return pltpu.make_async_copy(
            mem_hbm.at[:, pl.ds(lo + c * CH1, CH1)], buf, sem)

    def main_chunks(cur):
        cin(0, bufs[0], sems[0]).start()

        @pl.when(nch > 1)
        def _():
            cin(1, bufs[1], sems[1]).start()

        def step(c, cur, i):
            @pl.when(c + 2 < nch)
            def _():
                j = (i + 2) % 3
                cin(c + 2, bufs[j], sems[j]).start()

            cin(c, bufs[i], sems[i]).wait()
            return extract(bufs[i], c * CH1, CH1, cur)

        def body(c, cur):
            m = c % 3
            return lax.cond(
                m == 0, lambda: step(c, cur, 0),
                lambda: lax.cond(m == 1, lambda: step(c, cur, 1),
                                 lambda: step(c, cur, 2)))

        return lax.fori_loop(0, nch, body, cur)

    cur = lax.cond(nch > 0, lambda: main_chunks(jnp.int32(0)),
                   lambda: jnp.int32(0))

    def rem_body(r, cur):
        off = nch * CH1 + r * 128
        pltpu.async_copy(
            mem_hbm.at[:, pl.ds(lo + off, 128)], rb_v, sem_a).wait()
        return extract(rb_v, off, 128, cur)

    cur = lax.fori_loop(0, nrem, rem_body, cur)

    @pl.when(wid == NW - 1)
    def _():
        pltpu.sync_copy(tail_hbm, tb_v)
        extract(tb_v, span, NTAIL, cur)

    def dup_blk(b, _):
        pltpu.sync_copy(ids_hbm.at[pl.ds(b * IDS_BLK, IDS_BLK)], idsb_v)

        def dup_vec(i, _):
            ids = idsb_v[pl.ds(i * L, L)]
            pos = iota + (b * IDS_BLK + i * L)
            rel = ids - lo
            m = (rel >= 0) & (rel < span_t)
            idx = jnp.where(m, rel, 0)
            own = plsc.load_gather(own_v, [idx], mask=m)
            dup = m & (own != pos)
            nd = _popcnt(dup)

            dupi = jnp.where(dup, 1, 0)

            @pl.when(nd > 0)
            def _():
                for l in range(L):
                    sel = dupi[l] == 1

                    @pl.when(sel)
                    def _():
                        pltpu.async_copy(
                            h_hbm.at[pl.ds(own[l] * D, D)],
                            stg_v.at[pl.ds(l * 2 * L, 2 * L)],
                            sem_h,
                        )

                def drain(_k, _x):
                    pltpu.make_async_copy(
                        h_hbm.at[pl.ds(0, D)], stg_v.at[pl.ds(0, D)], sem_h
                    ).wait()
                    return 0

                lax.fori_loop(0, nd, drain, 0)

                for l in range(L):
                    sel = dupi[l] == 1

                    @pl.when(sel)
                    def _():
                        pltpu.async_copy(
                            stg_v.at[pl.ds(l * 2 * L, 2 * L)],
                            h_hbm.at[pl.ds(pos[l] * D, D)],
                            sem_h,
                        )

                lax.fori_loop(0, nd, drain, 0)

            return 0

        lax.fori_loop(0, IDS_BLK // L, dup_vec, 0)
        return 0

    lax.fori_loop(0, B // IDS_BLK, dup_blk, 0)


def _tc_body(h_ref, ts_ref, tw_ref, tb_ref, mw_ref, mb_ref, wih_ref, whh_ref,
             rb_ref, w1_ref, b1_ref, w2_ref, b2_ref, w3_ref, b3_ref,
             newh_ref, score_ref):
    h = h_ref[...]
    te = jnp.cos(ts_ref[...] * tw_ref[...] + tb_ref[...])
    msg = jnp.maximum(
        h @ mw_ref[0:D, :] + te @ mw_ref[D:2 * D, :] + mb_ref[...], 0.0)
    nh = jnp.tanh(msg @ wih_ref[...] + h @ whh_ref[...] + rb_ref[...])
    newh_ref[...] = nh
    x = jnp.maximum(
        h @ w1_ref[0:D, :] + nh @ w1_ref[D:2 * D, :] + b1_ref[...], 0.0)
    x = jnp.maximum(x @ w2_ref[...] + b2_ref[...], 0.0)
    score_ref[...] = (x @ w3_ref[...] + b3_ref[...])[:, 0]


_BLK = 2048


def _tc_compute(h, ts2, tw, tb, mw, mb, wih, whh, rb, w1, b1, w2, b2, w3, b3):
    full = lambda shape: pl.BlockSpec(shape, lambda i: (0, 0))
    return pl.pallas_call(
        _tc_body,
        grid=(B // _BLK,),
        in_specs=[
            pl.BlockSpec((_BLK, D), lambda i: (i, 0)),
            pl.BlockSpec((_BLK, 1), lambda i: (i, 0)),
            full((1, D)), full((1, D)),
            full((2 * D, D)), full((1, D)),
            full((D, D)), full((D, D)), full((1, D)),
            full((2 * D, 64)), full((1, 64)),
            full((64, 16)), full((1, 16)),
            full((16, 1)), full((1, 1)),
        ],
        out_specs=[
            pl.BlockSpec((_BLK, D), lambda i: (i, 0)),
            pl.BlockSpec((_BLK,), lambda i: (i,)),
        ],
        out_shape=[
            jax.ShapeDtypeStruct((B, D), jnp.float32),
            jax.ShapeDtypeStruct((B,), jnp.float32),
        ],
    )(h, ts2, tw, tb, mw, mb, wih, whh, rb, w1, b1, w2, b2, w3, b3)


@functools.partial(
    pl.kernel,
    mesh=_mesh,
    out_type=(
        jax.ShapeDtypeStruct((D, N), jnp.float32),
        jax.ShapeDtypeStruct((D, NTAIL), jnp.float32),
    ),
    compiler_params=_params,
    scratch_types=[
        pltpu.VMEM((B + L,), jnp.int32),
        pltpu.VMEM((B + L,), jnp.int32),
        pltpu.VMEM((STG,), jnp.float32),
        pltpu.VMEM((D, CHN), jnp.float32),
        pltpu.VMEM((D, CHN), jnp.float32),
        pltpu.VMEM((D, CHN), jnp.float32),
        pltpu.VMEM((D, CHN), jnp.float32),
        pltpu.VMEM((D, 128), jnp.float32),
        pltpu.VMEM((D, NTAIL), jnp.float32),
        pltpu.SemaphoreType.DMA,
        pltpu.SemaphoreType.DMA,
        pltpu.SemaphoreType.DMA,
        pltpu.SemaphoreType.DMA,
        pltpu.SemaphoreType.DMA,
    ],
)
def _sc_scatter_copy(mem_hbm, newh_hbm, tail_hbm, wrelx_hbm, wposx_hbm,
                     out_hbm, otail_hbm, wrel_v, wpos_v, stg_v,
                     cb0_v, cb1_v, cb2_v, cb3_v, rb_v, tb_v,
                     sem_a, sem_b, sem_c, sem_d, sem_w):
    wid = lax.axis_index("s") * NC + lax.axis_index("c")
    lo, span = _wrange(wid)
    iota = lax.iota(jnp.int32, L)

    pltpu.sync_copy(wrelx_hbm.at[pl.ds(wid * (B + L), B + L)], wrel_v)
    pltpu.sync_copy(wposx_hbm.at[pl.ds(wid * (B + L), B + L)], wpos_v)

    def patch(bufview, crel, csz, cur):
        chi = crel + csz

        def cond(cur):
            return wrel_v[pl.ds(cur, L)][0] < chi

        def wbody(cur):
            relv = wrel_v[pl.ds(cur, L)]
            posv = wpos_v[pl.ds(cur, L)]
            k = _popcnt(relv < chi)
            for l in range(L):
                @pl.when(l < k)
                def _():
                    pltpu.async_copy(
                        newh_hbm.at[pl.ds(posv[l] * D, D)],
                        stg_v.at[pl.ds(l * 2 * L, 2 * L)],
                        sem_w,
                    )

            def drain(_k, _x):
                pltpu.make_async_copy(
                    newh_hbm.at[pl.ds(0, D)], stg_v.at[pl.ds(0, D)], sem_w
                ).wait()
                return 0

            lax.fori_loop(0, k, drain, 0)

            for l in range(L):
                @pl.when(l < k)
                def _():
                    col = jnp.full((L,), relv[l] - crel, jnp.int32)
                    v0 = stg_v[pl.ds(l * 2 * L, L)]
                    v1 = stg_v[pl.ds(l * 2 * L + L, L)]
                    plsc.store_scatter(bufview, [iota, col], v0)
                    plsc.store_scatter(bufview, [iota + L, col], v1)

            return cur + k

        return lax.while_loop(cond, wbody, cur)

    nch = span // CHN
    nrem = (span - nch * CHN) // 128
    bufs = (cb0_v, cb1_v, cb2_v, cb3_v)
    sems = (sem_a, sem_b, sem_c, sem_d)

    def cin(c, buf, sem):
        return pltpu.make_async_copy(
            mem_hbm.at[:, pl.ds(lo + c * CHN, CHN)], buf, sem)

    def cout(c, buf, sem):
        return pltpu.make_async_copy(
            buf, out_hbm.at[:, pl.ds(lo + c * CHN, CHN)], sem)

    def main_chunks(cur):
        cin(0, bufs[0], sems[0]).start()
        cin(1, bufs[1], sems[1]).start()

        def step(c, cur, i):
            j = (i + 2) % 4

            @pl.when(c + 2 < nch)
            def _():
                @pl.when(c >= 2)
                def _():
                    cout(c - 2, bufs[j], sems[j]).wait()

                cin(c + 2, bufs[j], sems[j]).start()

            cin(c, bufs[i], sems[i]).wait()
            ncur = patch(bufs[i], c * CHN, CHN, cur)
            cout(c, bufs[i], sems[i]).start()
            return ncur

        def body(c, cur):
            m = c % 4
            return lax.cond(
                m < 2,
                lambda: lax.cond(m == 0, lambda: step(c, cur, 0),
                                 lambda: step(c, cur, 1)),
                lambda: lax.cond(m == 2, lambda: step(c, cur, 2),
                                 lambda: step(c, cur, 3)))

        cur = lax.fori_loop(0, nch, body, cur)

        def wait_out_at(k):
            for j in range(4):
                @pl.when(k % 4 == j)
                def _():
                    cout(k, bufs[j], sems[j]).wait()

        wait_out_at(nch - 4)
        wait_out_at(nch - 3)
        wait_out_at(nch - 2)
        wait_out_at(nch - 1)
        return cur

    cur = main_chunks(jnp.int32(0))

    def rem_body(r, cur):
        off = nch * CHN + r * 128
        pltpu.async_copy(
            mem_hbm.at[:, pl.ds(lo + off, 128)], rb_v, sem_a).wait()
        cur = patch(rb_v, off, 128, cur)
        pltpu.async_copy(
            rb_v, out_hbm.at[:, pl.ds(lo + off, 128)], sem_a).wait()
        return cur

    cur = lax.fori_loop(0, nrem, rem_body, cur)

    @pl.when(wid == NW - 1)
    def _():
        pltpu.sync_copy(tail_hbm, tb_v)
        patch(tb_v, span, NTAIL, cur)
        pltpu.sync_copy(tb_v, otail_hbm)


def kernel(memory, node_ids, timestamps, time_w, time_b, msg_W, msg_b,
           rnn_Wih, rnn_Whh, rnn_b, dec_W1, dec_b1, dec_W2, dec_b2,
           dec_W3, dec_b3):
    ids = node_ids.astype(jnp.int32)
    mem_t = memory.T
    mem_tail = lax.slice(mem_t, (0, TAIL), (D, N))
    h_flat, wrelx, wposx = _sc_gather(mem_t, ids, mem_tail)
    h = h_flat.reshape(B, D)
    newh, score = _tc_compute(
        h, timestamps.reshape(B, 1), time_w.reshape(1, D),
        time_b.reshape(1, D), msg_W, msg_b.reshape(1, D), rnn_Wih,
        rnn_Whh, rnn_b.reshape(1, D), dec_W1, dec_b1.reshape(1, 64),
        dec_W2, dec_b2.reshape(1, 16), dec_W3, dec_b3.reshape(1, 1))
    newh_flat = newh.reshape(B * D)
    out_main, out_tail = _sc_scatter_copy(mem_t, newh_flat, mem_tail,
                                          wrelx, wposx)
    out_t = lax.dynamic_update_slice(out_main, out_tail, (0, TAIL))
    return (out_t.T, score.reshape(B, 1))
